# baseline (speedup 1.0000x reference)
import functools

import jax
import jax.numpy as jnp
from jax import lax
from jax.experimental import pallas as pl
from jax.experimental.pallas import tpu as pltpu
from jax.experimental.pallas import tpu_sc as plsc

N_ROWS = 1000000
F = 32
B = 16384
H = 8
_PAD = 128
_ROWS4 = N_ROWS // 4

_NC = 2
_NS = 16
_NW = _NC * _NS
_BPW = B // _NW
_CH = 128
_NCHUNK = _BPW // _CH
_L = 16


def _extract_rows(buf, loff_v, stage, off):
    for g in range(_CH // _L):
        svec = loff_v[pl.ds(off + g * _L, _L)]
        for r in range(_L):
            row = g * _L + r
            s = svec[r]
            stage[row, pl.ds(0, _L)] = buf[row, pl.ds(s, _L)]
            stage[row, pl.ds(_L, _L)] = buf[row, pl.ds(s + _L, _L)]


def _gather_body(user_hbm, item_hbm, uf_hbm, if_hbm, ue_out, ie_out,
                 uidx_v, iidx_v, uloff_v, iloff_v, ubuf, ibuf,
                 ustage, istage, sem_u, sem_i):
    wid = lax.axis_index("s") * _NC + lax.axis_index("c")
    base = wid * _BPW
    pltpu.sync_copy(user_hbm.at[pl.ds(base, _BPW)], uidx_v)
    pltpu.sync_copy(item_hbm.at[pl.ds(base, _BPW)], iidx_v)
    for g in range(_BPW // _L):
        sl = pl.ds(g * _L, _L)
        uv = uidx_v[sl]
        uloff_v[sl] = jnp.bitwise_and(uv, 3) * F
        uidx_v[sl] = jnp.right_shift(uv, 2)
        iv = iidx_v[sl]
        iloff_v[sl] = jnp.bitwise_and(iv, 3) * F
        iidx_v[sl] = jnp.right_shift(iv, 2)
    for k in range(_NCHUNK):
        off = k * _CH
        cu = pltpu.async_copy(uf_hbm.at[uidx_v.at[pl.ds(off, _CH)]], ubuf, sem_u)
        ci = pltpu.async_copy(if_hbm.at[iidx_v.at[pl.ds(off, _CH)]], ibuf, sem_i)
        cu.wait()
        _extract_rows(ubuf, uloff_v, ustage, off)
        ci.wait()
        _extract_rows(ibuf, iloff_v, istage, off)
        pltpu.sync_copy(ustage, ue_out.at[pl.ds(base + off, _CH)])
        pltpu.sync_copy(istage, ie_out.at[pl.ds(base + off, _CH)])


_sc_gather = functools.partial(
    pl.kernel,
    out_type=[
        jax.ShapeDtypeStruct((B, _PAD), jnp.float32),
        jax.ShapeDtypeStruct((B, _PAD), jnp.float32),
    ],
    mesh=plsc.VectorSubcoreMesh(core_axis_name="c", subcore_axis_name="s"),
    scratch_types=[
        pltpu.VMEM((_BPW,), jnp.int32),
        pltpu.VMEM((_BPW,), jnp.int32),
        pltpu.VMEM((_BPW,), jnp.int32),
        pltpu.VMEM((_BPW,), jnp.int32),
        pltpu.VMEM((_CH, _PAD), jnp.float32),
        pltpu.VMEM((_CH, _PAD), jnp.float32),
        pltpu.VMEM((_CH, _PAD), jnp.float32),
        pltpu.VMEM((_CH, _PAD), jnp.float32),
        pltpu.SemaphoreType.DMA,
        pltpu.SemaphoreType.DMA,
    ],
    compiler_params=pltpu.CompilerParams(needs_layout_passes=False),
)(_gather_body)


def _mlp_body(ue_ref, ie_ref, w1u_ref, w1i_ref, b1_ref, w3_ref, b3_ref, out_ref):
    ue = ue_ref[...][:, :F]
    ie = ie_ref[...][:, :F]
    h = (jnp.dot(ue, w1u_ref[...], preferred_element_type=jnp.float32)
         + jnp.dot(ie, w1i_ref[...], preferred_element_type=jnp.float32)
         + b1_ref[...])
    h = jnp.maximum(h, 0.0)
    z = jnp.dot(h, w3_ref[...], preferred_element_type=jnp.float32) + b3_ref[...]
    out_ref[...] = jax.nn.sigmoid(z)


_BLK = 4096


def _mlp(ue, ie, w1u, w1i, b1, w3, b3):
    grid = (B // _BLK,)
    return pl.pallas_call(
        _mlp_body,
        grid=grid,
        in_specs=[
            pl.BlockSpec((_BLK, _PAD), lambda i: (i, 0)),
            pl.BlockSpec((_BLK, _PAD), lambda i: (i, 0)),
            pl.BlockSpec((F, H), lambda i: (0, 0)),
            pl.BlockSpec((F, H), lambda i: (0, 0)),
            pl.BlockSpec((1, H), lambda i: (0, 0)),
            pl.BlockSpec((H, 1), lambda i: (0, 0)),
            pl.BlockSpec((1, 1), lambda i: (0, 0)),
        ],
        out_specs=pl.BlockSpec((_BLK, 1), lambda i: (i, 0)),
        out_shape=jax.ShapeDtypeStruct((B, 1), jnp.float32),
    )(ue, ie, w1u, w1i, b1, w3, b3)


def kernel(user, item, user_factors, item_factors, W1, b1, W3, b3):
    user = user.astype(jnp.int32)
    item = item.astype(jnp.int32)
    uf2 = user_factors.reshape(_ROWS4, _PAD)
    if2 = item_factors.reshape(_ROWS4, _PAD)
    ue, ie = _sc_gather(user, item, uf2, if2)
    return _mlp(ue, ie, W1[:F], W1[F:], b1.reshape(1, H), W3, b3.reshape(1, 1))


# speedup vs baseline: 1.3144x; 1.3144x over previous
import functools

import jax
import jax.numpy as jnp
from jax import lax
from jax.experimental import pallas as pl
from jax.experimental.pallas import tpu as pltpu
from jax.experimental.pallas import tpu_sc as plsc

N_ROWS = 1000000
F = 32
B = 16384
H = 8
_TILES = N_ROWS // 8

_NC = 2
_NS = 16
_NW = _NC * _NS
_BPW = B // _NW
_CH = 32
_NCHUNK = _BPW // _CH
_L = 16


def _extract_rows(buf, sub_v, stage, off):
    lanes = lax.iota(jnp.int32, _L)
    for g in range(_CH // _L):
        sub = sub_v[pl.ds(off + g * _L, _L)]
        li = lanes + g * _L
        for c in range(F):
            cc = jnp.full((_L,), c, jnp.int32)
            vals = plsc.load_gather(buf, [li, sub, cc])
            plsc.store_scatter(stage, [li, cc], vals)


def _gather_body(user_hbm, item_hbm, uf_hbm, if_hbm, ue_out, ie_out,
                 uidx_v, iidx_v, usub_v, isub_v, ubuf, ibuf,
                 ustage, istage, sem_u, sem_i):
    wid = lax.axis_index("s") * _NC + lax.axis_index("c")
    base = wid * _BPW
    pltpu.sync_copy(user_hbm.at[pl.ds(base, _BPW)], uidx_v)
    pltpu.sync_copy(item_hbm.at[pl.ds(base, _BPW)], iidx_v)
    for g in range(_BPW // _L):
        sl = pl.ds(g * _L, _L)
        uv = uidx_v[sl]
        usub_v[sl] = jnp.bitwise_and(uv, 7)
        uidx_v[sl] = jnp.bitwise_and(uv, -8)
        iv = iidx_v[sl]
        isub_v[sl] = jnp.bitwise_and(iv, 7)
        iidx_v[sl] = jnp.bitwise_and(iv, -8)

    def chunk_body(k, _):
        off = k * _CH
        copies = []
        for g in range(_CH // _L):
            ut_vec = uidx_v[pl.ds(off + g * _L, _L)]
            it_vec = iidx_v[pl.ds(off + g * _L, _L)]
            for r in range(_L):
                row = g * _L + r
                ut = pl.multiple_of(ut_vec[r], 8)
                it = pl.multiple_of(it_vec[r], 8)
                copies.append(pltpu.async_copy(
                    uf_hbm.at[pl.ds(ut, 8)], ubuf.at[row], sem_u))
                copies.append(pltpu.async_copy(
                    if_hbm.at[pl.ds(it, 8)], ibuf.at[row], sem_i))
        for cp in copies:
            cp.wait()
        _extract_rows(ubuf, usub_v, ustage, off)
        _extract_rows(ibuf, isub_v, istage, off)
        pltpu.sync_copy(ustage, ue_out.at[pl.ds(base + off, _CH)])
        pltpu.sync_copy(istage, ie_out.at[pl.ds(base + off, _CH)])
        return ()

    lax.fori_loop(0, _NCHUNK, chunk_body, (), unroll=False)


_sc_gather = functools.partial(
    pl.kernel,
    out_type=[
        jax.ShapeDtypeStruct((B, F), jnp.float32),
        jax.ShapeDtypeStruct((B, F), jnp.float32),
    ],
    mesh=plsc.VectorSubcoreMesh(core_axis_name="c", subcore_axis_name="s"),
    scratch_types=[
        pltpu.VMEM((_BPW,), jnp.int32),
        pltpu.VMEM((_BPW,), jnp.int32),
        pltpu.VMEM((_BPW,), jnp.int32),
        pltpu.VMEM((_BPW,), jnp.int32),
        pltpu.VMEM((_CH, 8, F), jnp.float32),
        pltpu.VMEM((_CH, 8, F), jnp.float32),
        pltpu.VMEM((_CH, F), jnp.float32),
        pltpu.VMEM((_CH, F), jnp.float32),
        pltpu.SemaphoreType.DMA,
        pltpu.SemaphoreType.DMA,
    ],
    compiler_params=pltpu.CompilerParams(needs_layout_passes=False),
)(_gather_body)


def _mlp_body(ue_ref, ie_ref, w1u_ref, w1i_ref, b1_ref, w3_ref, b3_ref, out_ref):
    h = (jnp.dot(ue_ref[...], w1u_ref[...], preferred_element_type=jnp.float32)
         + jnp.dot(ie_ref[...], w1i_ref[...], preferred_element_type=jnp.float32)
         + b1_ref[...])
    h = jnp.maximum(h, 0.0)
    z = jnp.dot(h, w3_ref[...], preferred_element_type=jnp.float32) + b3_ref[...]
    out_ref[...] = jax.nn.sigmoid(z)


_BLK = 4096


def _mlp(ue, ie, w1u, w1i, b1, w3, b3):
    grid = (B // _BLK,)
    return pl.pallas_call(
        _mlp_body,
        grid=grid,
        in_specs=[
            pl.BlockSpec((_BLK, F), lambda i: (i, 0)),
            pl.BlockSpec((_BLK, F), lambda i: (i, 0)),
            pl.BlockSpec((F, H), lambda i: (0, 0)),
            pl.BlockSpec((F, H), lambda i: (0, 0)),
            pl.BlockSpec((1, H), lambda i: (0, 0)),
            pl.BlockSpec((H, 1), lambda i: (0, 0)),
            pl.BlockSpec((1, 1), lambda i: (0, 0)),
        ],
        out_specs=pl.BlockSpec((_BLK, 1), lambda i: (i, 0)),
        out_shape=jax.ShapeDtypeStruct((B, 1), jnp.float32),
    )(ue, ie, w1u, w1i, b1, w3, b3)


def kernel(user, item, user_factors, item_factors, W1, b1, W3, b3):
    user = user.astype(jnp.int32)
    item = item.astype(jnp.int32)
    ue, ie = _sc_gather(user, item, user_factors, item_factors)
    return _mlp(ue, ie, W1[:F], W1[F:], b1.reshape(1, H), W3, b3.reshape(1, 1))


# speedup vs baseline: 2.0581x; 1.5658x over previous
import functools

import jax
import jax.numpy as jnp
from jax import lax
from jax.experimental import pallas as pl
from jax.experimental.pallas import tpu as pltpu
from jax.experimental.pallas import tpu_sc as plsc

N_ROWS = 1000000
F = 32
B = 16384
H = 8
_TILES = N_ROWS // 8

_NC = 2
_NS = 16
_NW = _NC * _NS
_BPW = B // _NW
_CH = 16
_NCHUNK = _BPW // _CH
_L = 16


def _extract_rows(buf, sub_v, stage, off):
    lanes = lax.iota(jnp.int32, _L)
    for g in range(_CH // _L):
        sub = sub_v[pl.ds(off + g * _L, _L)]
        li = lanes + g * _L
        for c in range(F):
            cc = jnp.full((_L,), c, jnp.int32)
            vals = plsc.load_gather(buf, [li, sub, cc])
            plsc.store_scatter(stage, [li, cc], vals)


def _gather_body(user_hbm, item_hbm, uf_hbm, if_hbm, ue_out, ie_out,
                 uidx_v, iidx_v, usub_v, isub_v,
                 ubuf0, ibuf0, ubuf1, ibuf1, ustage, istage, sem0, sem1):
    wid = lax.axis_index("s") * _NC + lax.axis_index("c")
    base = wid * _BPW
    pltpu.sync_copy(user_hbm.at[pl.ds(base, _BPW)], uidx_v)
    pltpu.sync_copy(item_hbm.at[pl.ds(base, _BPW)], iidx_v)
    for g in range(_BPW // _L):
        sl = pl.ds(g * _L, _L)
        uv = uidx_v[sl]
        usub_v[sl] = jnp.bitwise_and(uv, 7)
        uidx_v[sl] = jnp.right_shift(uv, 3)
        iv = iidx_v[sl]
        isub_v[sl] = jnp.bitwise_and(iv, 7)
        iidx_v[sl] = jnp.right_shift(iv, 3)

    def issue(off, ub, ib, sem):
        for g in range(_CH // _L):
            ut_vec = uidx_v[pl.ds(off + g * _L, _L)]
            it_vec = iidx_v[pl.ds(off + g * _L, _L)]
            for r in range(_L):
                row = g * _L + r
                pltpu.async_copy(uf_hbm.at[ut_vec[r]], ub.at[row], sem)
                pltpu.async_copy(if_hbm.at[it_vec[r]], ib.at[row], sem)

    def drain(ub, ib, sem):
        pltpu.make_async_copy(uf_hbm.at[pl.ds(0, _CH)], ub, sem).wait()
        pltpu.make_async_copy(if_hbm.at[pl.ds(0, _CH)], ib, sem).wait()

    def finish(off, ub, ib):
        _extract_rows(ub, usub_v, ustage, off)
        _extract_rows(ib, isub_v, istage, off)
        pltpu.sync_copy(ustage, ue_out.at[pl.ds(base + off, _CH)])
        pltpu.sync_copy(istage, ie_out.at[pl.ds(base + off, _CH)])

    def chunk_body(j, _):
        off = j * (2 * _CH)
        issue(off, ubuf0, ibuf0, sem0)
        issue(off + _CH, ubuf1, ibuf1, sem1)
        drain(ubuf0, ibuf0, sem0)
        finish(off, ubuf0, ibuf0)
        drain(ubuf1, ibuf1, sem1)
        finish(off + _CH, ubuf1, ibuf1)
        return ()

    lax.fori_loop(0, _NCHUNK // 2, chunk_body, (), unroll=False)


_sc_gather = functools.partial(
    pl.kernel,
    out_type=[
        jax.ShapeDtypeStruct((B, F), jnp.float32),
        jax.ShapeDtypeStruct((B, F), jnp.float32),
    ],
    mesh=plsc.VectorSubcoreMesh(core_axis_name="c", subcore_axis_name="s"),
    scratch_types=[
        pltpu.VMEM((_BPW,), jnp.int32),
        pltpu.VMEM((_BPW,), jnp.int32),
        pltpu.VMEM((_BPW,), jnp.int32),
        pltpu.VMEM((_BPW,), jnp.int32),
        pltpu.VMEM((_CH, 8, F), jnp.float32),
        pltpu.VMEM((_CH, 8, F), jnp.float32),
        pltpu.VMEM((_CH, 8, F), jnp.float32),
        pltpu.VMEM((_CH, 8, F), jnp.float32),
        pltpu.VMEM((_CH, F), jnp.float32),
        pltpu.VMEM((_CH, F), jnp.float32),
        pltpu.SemaphoreType.DMA,
        pltpu.SemaphoreType.DMA,
    ],
    compiler_params=pltpu.CompilerParams(needs_layout_passes=False),
)(_gather_body)


def _mlp_body(ue_ref, ie_ref, w1u_ref, w1i_ref, b1_ref, w3_ref, b3_ref, out_ref):
    h = (jnp.dot(ue_ref[...], w1u_ref[...], preferred_element_type=jnp.float32)
         + jnp.dot(ie_ref[...], w1i_ref[...], preferred_element_type=jnp.float32)
         + b1_ref[...])
    h = jnp.maximum(h, 0.0)
    z = jnp.dot(h, w3_ref[...], preferred_element_type=jnp.float32) + b3_ref[...]
    out_ref[...] = jax.nn.sigmoid(z)


_BLK = 4096


def _mlp(ue, ie, w1u, w1i, b1, w3, b3):
    grid = (B // _BLK,)
    return pl.pallas_call(
        _mlp_body,
        grid=grid,
        in_specs=[
            pl.BlockSpec((_BLK, F), lambda i: (i, 0)),
            pl.BlockSpec((_BLK, F), lambda i: (i, 0)),
            pl.BlockSpec((F, H), lambda i: (0, 0)),
            pl.BlockSpec((F, H), lambda i: (0, 0)),
            pl.BlockSpec((1, H), lambda i: (0, 0)),
            pl.BlockSpec((H, 1), lambda i: (0, 0)),
            pl.BlockSpec((1, 1), lambda i: (0, 0)),
        ],
        out_specs=pl.BlockSpec((_BLK, 1), lambda i: (i, 0)),
        out_shape=jax.ShapeDtypeStruct((B, 1), jnp.float32),
    )(ue, ie, w1u, w1i, b1, w3, b3)


def kernel(user, item, user_factors, item_factors, W1, b1, W3, b3):
    user = user.astype(jnp.int32)
    item = item.astype(jnp.int32)
    uf3 = user_factors.reshape(_TILES, 8, F)
    if3 = item_factors.reshape(_TILES, 8, F)
    ue, ie = _sc_gather(user, item, uf3, if3)
    return _mlp(ue, ie, W1[:F], W1[F:], b1.reshape(1, H), W3, b3.reshape(1, 1))


# speedup vs baseline: 2.0601x; 1.0010x over previous
import functools

import jax
import jax.numpy as jnp
from jax import lax
from jax.experimental import pallas as pl
from jax.experimental.pallas import tpu as pltpu
from jax.experimental.pallas import tpu_sc as plsc

N_ROWS = 1000000
F = 32
B = 16384
H = 8
_TILES = N_ROWS // 8

_NC = 2
_NS = 16
_NW = _NC * _NS
_BPW = B // _NW
_CH = 32
_NCHUNK = _BPW // _CH
_L = 16


def _extract_rows(buf, sub_v, stage, off):
    lanes = lax.iota(jnp.int32, _L)
    for g in range(_CH // _L):
        sub = sub_v[pl.ds(off + g * _L, _L)]
        li = lanes + g * _L
        for c in range(F):
            cc = jnp.full((_L,), c, jnp.int32)
            vals = plsc.load_gather(buf, [li, sub, cc])
            plsc.store_scatter(stage, [li, cc], vals)


def _gather_body(user_hbm, item_hbm, uf_hbm, if_hbm, ue_out, ie_out,
                 uidx_v, iidx_v, usub_v, isub_v,
                 ubuf0, ibuf0, ustage, istage, sem0):
    wid = lax.axis_index("s") * _NC + lax.axis_index("c")
    base = wid * _BPW
    pltpu.sync_copy(user_hbm.at[pl.ds(base, _BPW)], uidx_v)
    pltpu.sync_copy(item_hbm.at[pl.ds(base, _BPW)], iidx_v)
    for g in range(_BPW // _L):
        sl = pl.ds(g * _L, _L)
        uv = uidx_v[sl]
        usub_v[sl] = jnp.bitwise_and(uv, 7)
        uidx_v[sl] = jnp.right_shift(uv, 3)
        iv = iidx_v[sl]
        isub_v[sl] = jnp.bitwise_and(iv, 7)
        iidx_v[sl] = jnp.right_shift(iv, 3)

    def issue(off, ub, ib, sem):
        for g in range(_CH // _L):
            ut_vec = uidx_v[pl.ds(off + g * _L, _L)]
            it_vec = iidx_v[pl.ds(off + g * _L, _L)]
            for r in range(_L):
                row = g * _L + r
                pltpu.async_copy(uf_hbm.at[ut_vec[r]], ub.at[row], sem)
                pltpu.async_copy(if_hbm.at[it_vec[r]], ib.at[row], sem)

    def drain(ub, ib, sem):
        pltpu.make_async_copy(uf_hbm.at[pl.ds(0, _CH)], ub, sem).wait()
        pltpu.make_async_copy(if_hbm.at[pl.ds(0, _CH)], ib, sem).wait()

    def finish(off, ub, ib):
        _extract_rows(ub, usub_v, ustage, off)
        _extract_rows(ib, isub_v, istage, off)
        pltpu.sync_copy(ustage, ue_out.at[pl.ds(base + off, _CH)])
        pltpu.sync_copy(istage, ie_out.at[pl.ds(base + off, _CH)])

    def chunk_body(j, _):
        off = j * _CH
        issue(off, ubuf0, ibuf0, sem0)
        drain(ubuf0, ibuf0, sem0)
        finish(off, ubuf0, ibuf0)
        return ()

    lax.fori_loop(0, _NCHUNK, chunk_body, (), unroll=False)


_sc_gather = functools.partial(
    pl.kernel,
    out_type=[
        jax.ShapeDtypeStruct((B, F), jnp.float32),
        jax.ShapeDtypeStruct((B, F), jnp.float32),
    ],
    mesh=plsc.VectorSubcoreMesh(core_axis_name="c", subcore_axis_name="s"),
    scratch_types=[
        pltpu.VMEM((_BPW,), jnp.int32),
        pltpu.VMEM((_BPW,), jnp.int32),
        pltpu.VMEM((_BPW,), jnp.int32),
        pltpu.VMEM((_BPW,), jnp.int32),
        pltpu.VMEM((_CH, 8, F), jnp.float32),
        pltpu.VMEM((_CH, 8, F), jnp.float32),
        pltpu.VMEM((_CH, F), jnp.float32),
        pltpu.VMEM((_CH, F), jnp.float32),
        pltpu.SemaphoreType.DMA,
    ],
    compiler_params=pltpu.CompilerParams(needs_layout_passes=False),
)(_gather_body)


def _mlp_body(ue_ref, ie_ref, w1u_ref, w1i_ref, b1_ref, w3_ref, b3_ref, out_ref):
    h = (jnp.dot(ue_ref[...], w1u_ref[...], preferred_element_type=jnp.float32)
         + jnp.dot(ie_ref[...], w1i_ref[...], preferred_element_type=jnp.float32)
         + b1_ref[...])
    h = jnp.maximum(h, 0.0)
    z = jnp.dot(h, w3_ref[...], preferred_element_type=jnp.float32) + b3_ref[...]
    out_ref[...] = jax.nn.sigmoid(z)


_BLK = 4096


def _mlp(ue, ie, w1u, w1i, b1, w3, b3):
    grid = (B // _BLK,)
    return pl.pallas_call(
        _mlp_body,
        grid=grid,
        in_specs=[
            pl.BlockSpec((_BLK, F), lambda i: (i, 0)),
            pl.BlockSpec((_BLK, F), lambda i: (i, 0)),
            pl.BlockSpec((F, H), lambda i: (0, 0)),
            pl.BlockSpec((F, H), lambda i: (0, 0)),
            pl.BlockSpec((1, H), lambda i: (0, 0)),
            pl.BlockSpec((H, 1), lambda i: (0, 0)),
            pl.BlockSpec((1, 1), lambda i: (0, 0)),
        ],
        out_specs=pl.BlockSpec((_BLK, 1), lambda i: (i, 0)),
        out_shape=jax.ShapeDtypeStruct((B, 1), jnp.float32),
    )(ue, ie, w1u, w1i, b1, w3, b3)


def kernel(user, item, user_factors, item_factors, W1, b1, W3, b3):
    user = user.astype(jnp.int32)
    item = item.astype(jnp.int32)
    uf3 = user_factors.reshape(_TILES, 8, F)
    if3 = item_factors.reshape(_TILES, 8, F)
    ue, ie = _sc_gather(user, item, uf3, if3)
    return _mlp(ue, ie, W1[:F], W1[F:], b1.reshape(1, H), W3, b3.reshape(1, 1))
